# Initial kernel scaffold; baseline (speedup 1.0000x reference)
import jax
import jax.numpy as jnp
from jax import lax
from jax.experimental import pallas as pl
from jax.experimental.pallas import tpu as pltpu
from jax.experimental.pallas import tpu_sc as plsc

_EMBED = 64
_B = 16384 * 50
_NC = 2
_NS = 16
_NW = _NC * _NS
_BPW = _B // _NW
_C = 1024
_NCHUNK = _BPW // _C


def _gather_body(idx_hbm, table_hbm, out_hbm, idx_v, rows_v, sem):
    wid = lax.axis_index("s") * _NC + lax.axis_index("c")
    base = wid * _BPW

    def step(i, carry):
        off = base + i * _C
        pltpu.sync_copy(idx_hbm.at[pl.ds(off, _C)], idx_v)
        pltpu.async_copy(table_hbm.at[idx_v], rows_v, sem).wait()
        pltpu.sync_copy(rows_v, out_hbm.at[pl.ds(off, _C)])
        return carry

    lax.fori_loop(0, _NCHUNK, step, 0)


@jax.jit
def kernel(token_idx, tokenvectors):
    n0, n1 = token_idx.shape
    idx = token_idx.reshape(-1).astype(jnp.int32)
    run = pl.kernel(
        _gather_body,
        out_type=jax.ShapeDtypeStruct((_B, _EMBED), jnp.float32),
        mesh=plsc.VectorSubcoreMesh(core_axis_name="c", subcore_axis_name="s"),
        scratch_types=[
            pltpu.VMEM((_C,), jnp.int32),
            pltpu.VMEM((_C, _EMBED), jnp.float32),
            pltpu.SemaphoreType.DMA,
        ],
    )
    out = run(idx, tokenvectors)
    return out.reshape(n0, n1, _EMBED)


# speedup vs baseline: 1.8449x; 1.8449x over previous
import jax
import jax.numpy as jnp
from jax import lax
from jax.experimental import pallas as pl
from jax.experimental.pallas import tpu as pltpu
from jax.experimental.pallas import tpu_sc as plsc

_EMBED = 64
_B = 16384 * 50
_NC = 2
_NS = 16
_NW = _NC * _NS
_BPW = _B // _NW
_C = 1024
_NCHUNK = _BPW // _C


def _gather_body(idx_hbm, table_hbm, out_hbm, idx_v, rows_v, sem):
    wid = lax.axis_index("s") * _NC + lax.axis_index("c")
    base = wid * _BPW

    def step(i, carry):
        off = base + i * _C
        pltpu.sync_copy(idx_hbm.at[pl.ds(off, _C)], idx_v)
        pltpu.async_copy(table_hbm.at[idx_v], rows_v, sem).wait()
        pltpu.sync_copy(rows_v, out_hbm.at[pl.ds(off, _C)])
        return carry

    lax.fori_loop(0, _NCHUNK, step, 0)


@jax.jit
def kernel(token_idx, tokenvectors):
    n0, n1 = token_idx.shape
    idx = token_idx.reshape(-1).astype(jnp.int32)
    run = pl.kernel(
        _gather_body,
        out_type=jax.ShapeDtypeStruct((_B, _EMBED), jnp.float32),
        mesh=plsc.VectorSubcoreMesh(core_axis_name="c", subcore_axis_name="s"),
        scratch_types=[
            pltpu.VMEM((_C,), jnp.int32),
            pltpu.VMEM((_C, _EMBED), jnp.float32),
            pltpu.SemaphoreType.DMA,
        ],
        compiler_params=pltpu.CompilerParams(use_tc_tiling_on_sc=False),
    )
    out = run(idx, tokenvectors)
    return out.reshape(n0, n1, _EMBED)


# speedup vs baseline: 1.8838x; 1.0211x over previous
import jax
import jax.numpy as jnp
from jax import lax
from jax.experimental import pallas as pl
from jax.experimental.pallas import tpu as pltpu
from jax.experimental.pallas import tpu_sc as plsc

_EMBED = 64
_B = 16384 * 50
_NC = 2
_NS = 16
_NW = _NC * _NS
_BPW = _B // _NW
_C = 800
_N = _BPW // _C


def _gather_body(idx_hbm, table_hbm, out_hbm, idx_v, rows0, rows1, g0, g1, o0, o1):
    wid = lax.axis_index("s") * _NC + lax.axis_index("c")
    base = wid * _BPW

    rows = (rows0, rows1)
    gsem = (g0, g1)
    osem = (o0, o1)

    pltpu.sync_copy(idx_hbm.at[pl.ds(base, _BPW)], idx_v)

    gathers = [None] * _N
    outs = [None] * _N
    for g in range(_N):
        b = g % 2
        if g >= 2:
            outs[g - 2].wait()
        gathers[g] = pltpu.async_copy(
            table_hbm.at[idx_v.at[pl.ds(g * _C, _C)]], rows[b], gsem[b])
        if g >= 1:
            gathers[g - 1].wait()
            outs[g - 1] = pltpu.async_copy(
                rows[1 - b], out_hbm.at[pl.ds(base + (g - 1) * _C, _C)],
                osem[1 - b])
    gathers[_N - 1].wait()
    outs[_N - 1] = pltpu.async_copy(
        rows[(_N - 1) % 2], out_hbm.at[pl.ds(base + (_N - 1) * _C, _C)],
        osem[(_N - 1) % 2])
    outs[_N - 2].wait()
    outs[_N - 1].wait()


@jax.jit
def kernel(token_idx, tokenvectors):
    n0, n1 = token_idx.shape
    idx = token_idx.reshape(-1).astype(jnp.int32)
    run = pl.kernel(
        _gather_body,
        out_type=jax.ShapeDtypeStruct((_B, _EMBED), jnp.float32),
        mesh=plsc.VectorSubcoreMesh(core_axis_name="c", subcore_axis_name="s"),
        scratch_types=[
            pltpu.VMEM((_BPW,), jnp.int32),
            pltpu.VMEM((_C, _EMBED), jnp.float32),
            pltpu.VMEM((_C, _EMBED), jnp.float32),
            pltpu.SemaphoreType.DMA,
            pltpu.SemaphoreType.DMA,
            pltpu.SemaphoreType.DMA,
            pltpu.SemaphoreType.DMA,
        ],
        compiler_params=pltpu.CompilerParams(use_tc_tiling_on_sc=False),
    )
    out = run(idx, tokenvectors)
    return out.reshape(n0, n1, _EMBED)
